# baseline (speedup 1.0000x reference)
import jax
import jax.numpy as jnp
from jax.experimental import pallas as pl

B = 4096
D = 256


def _tail_body(a1w, a1b, b1r, w2t, b2, w3, b3, out):
    h = jnp.concatenate(
        [jnp.clip(a1w[0] + b1r[...], 0.0, 1.0),
         jnp.clip(a1b[0] + b1r[...], 0.0, 1.0)], axis=1)
    a2 = jnp.dot(h, w2t[...], preferred_element_type=jnp.float32) + b2[...]
    a2 = jnp.clip(a2, 0.0, 1.0)
    out[...] = jnp.sum(a2 * w3[...], axis=1, keepdims=True) + b3[...]


def _tail(acc, b1r, w2t, b2, w3, b3):
    R = 512
    return pl.pallas_call(
        _tail_body,
        grid=(B // R,),
        in_specs=[
            pl.BlockSpec((1, R, D), lambda i: (0, i, 0)),
            pl.BlockSpec((1, R, D), lambda i: (1, i, 0)),
            pl.BlockSpec((1, D), lambda i: (0, 0)),
            pl.BlockSpec((2 * D, 32), lambda i: (0, 0)),
            pl.BlockSpec((1, 32), lambda i: (0, 0)),
            pl.BlockSpec((1, 32), lambda i: (0, 0)),
            pl.BlockSpec((1, 1), lambda i: (0, 0)),
        ],
        out_specs=pl.BlockSpec((R, 1), lambda i: (i, 0)),
        out_shape=jax.ShapeDtypeStruct((B, 1), jnp.float32),
    )(acc, acc, b1r, w2t, b2, w3, b3)


def kernel(batch_size, white_features, white_indices, black_features,
           black_indices, W1, b1, W2, b2, W3, b3):
    bs_res = (jnp.asarray(batch_size) - B).astype(jnp.float32)
    w1t = W1.T

    def side(features, indices):
        f = jnp.take(w1t, features, axis=0)
        return jnp.zeros((B, D), jnp.float32).at[indices].add(f) + bs_res

    acc = jnp.stack([side(white_features, white_indices),
                     side(black_features, black_indices)])

    out = _tail(acc, (b1).reshape(1, D), W2.T, b2.reshape(1, 32), W3,
                b3.reshape(1, 1))
    return out[:, 0]


# speedup vs baseline: 2.4602x; 2.4602x over previous
import functools

import jax
import jax.numpy as jnp
from jax import lax
from jax.experimental import pallas as pl
from jax.experimental.pallas import tpu as pltpu
from jax.experimental.pallas import tpu_sc as plsc

B = 4096
D = 256
DH = D // 2
NF = 131072
NC, NS = 2, 16
CH = 64
GB = 64
NCH_TOT = NF // CH
ROWS_PER_SUB = B // NS


def _sc_body(w1t, feats, idxs, bnds, zblk, out, fbig, bbig, rows, bv, acc,
             gsem):
    c = lax.axis_index("c")
    s = lax.axis_index("s")

    def side(v):
        pltpu.sync_copy(zblk, acc)
        pltpu.sync_copy(bnds.at[v, s], bv)
        bvv = bv[...]
        lo = bvv[0]
        hi = bvv[1]
        a_lo = (lo // (8 * CH)) * (8 * CH)
        crow0 = a_lo // CH
        nch = (hi - a_lo + CH - 1) // CH

        def stage_bulk(g):
            gpar = lax.rem(g, 2)
            start = pl.multiple_of(crow0 + g * GB, 8)
            pltpu.sync_copy(feats.at[v, pl.ds(start, GB)], fbig.at[gpar])
            pltpu.sync_copy(idxs.at[v, pl.ds(start, GB)], bbig.at[gpar])

        def issue_gather(k):
            gpar = lax.rem(k // GB, 2)
            cpar = lax.rem(k, 2)
            pltpu.async_copy(w1t.at[c].at[fbig.at[gpar, lax.rem(k, GB)]],
                             rows.at[cpar], gsem)

        stage_bulk(0)

        @pl.when(nch > 0)
        def _():
            issue_gather(0)

        def chunk(k, carry):
            cpar = lax.rem(k, 2)
            gpar = lax.rem(k // GB, 2)
            km = lax.rem(k, GB)
            pltpu.make_async_copy(w1t.at[c].at[fbig.at[gpar, km]],
                                  rows.at[cpar], gsem).wait()

            @pl.when(k + 1 < nch)
            def _():
                @pl.when(lax.rem(k + 1, GB) == 0)
                def _():
                    stage_bulk((k + 1) // GB)
                issue_gather(k + 1)

            base = a_lo + k * CH

            def row16(t, carry2):
                r0 = t * 16
                li_vec = bbig[gpar, km, pl.ds(r0, 16)] - s * ROWS_PER_SUB

                for rr in range(16):
                    p = base + r0 + rr
                    li = li_vec[rr]

                    @pl.when(jnp.logical_and(p >= lo, p < hi))
                    def _(li=li, rr=rr):
                        for j in range(DH // 16):
                            sl = pl.ds(j * 16, 16)
                            acc[li, sl] += rows[cpar, r0 + rr, sl]

                return carry2

            lax.fori_loop(0, CH // 16, row16, 0)
            return carry

        lax.fori_loop(0, nch, chunk, 0)

        pltpu.sync_copy(
            acc,
            out.at[v, pl.ds(s * ROWS_PER_SUB, ROWS_PER_SUB),
                   pl.ds(c * DH, DH)])

    side(0)
    side(1)


_sc_accumulate = functools.partial(
    pl.kernel,
    out_type=jax.ShapeDtypeStruct((NC, B, D), jnp.float32),
    mesh=plsc.VectorSubcoreMesh(
        core_axis_name="c", subcore_axis_name="s",
        num_cores=NC, num_subcores=NS),
    scratch_types=[
        pltpu.VMEM((2, GB, CH), jnp.int32),
        pltpu.VMEM((2, GB, CH), jnp.int32),
        pltpu.VMEM((2, CH, DH), jnp.float32),
        pltpu.VMEM((16,), jnp.int32),
        pltpu.VMEM((ROWS_PER_SUB, DH), jnp.float32),
        pltpu.SemaphoreType.DMA,
    ],
)(_sc_body)


def _tail_body(a1w, a1b, b1r, w2t, b2, w3, b3, out):
    h = jnp.concatenate(
        [jnp.clip(a1w[0] + b1r[...], 0.0, 1.0),
         jnp.clip(a1b[0] + b1r[...], 0.0, 1.0)], axis=1)
    a2 = jnp.dot(h, w2t[...], preferred_element_type=jnp.float32) + b2[...]
    a2 = jnp.clip(a2, 0.0, 1.0)
    out[...] = jnp.sum(a2 * w3[...], axis=1, keepdims=True) + b3[...]


def _tail(acc, b1r, w2t, b2, w3, b3):
    R = 512
    return pl.pallas_call(
        _tail_body,
        grid=(B // R,),
        in_specs=[
            pl.BlockSpec((1, R, D), lambda i: (0, i, 0)),
            pl.BlockSpec((1, R, D), lambda i: (1, i, 0)),
            pl.BlockSpec((1, D), lambda i: (0, 0)),
            pl.BlockSpec((2 * D, 32), lambda i: (0, 0)),
            pl.BlockSpec((1, 32), lambda i: (0, 0)),
            pl.BlockSpec((1, 32), lambda i: (0, 0)),
            pl.BlockSpec((1, 1), lambda i: (0, 0)),
        ],
        out_specs=pl.BlockSpec((R, 1), lambda i: (i, 0)),
        out_shape=jax.ShapeDtypeStruct((B, 1), jnp.float32),
    )(acc, acc, b1r, w2t, b2, w3, b3)


def _bounds(indices):
    edges = jnp.arange(0, B + 1, ROWS_PER_SUB, dtype=jnp.int32)
    b = jnp.searchsorted(indices, edges, side="left").astype(jnp.int32)
    lohi = jnp.stack([b[:-1], b[1:]], axis=-1)
    return jnp.pad(lohi, ((0, 0), (0, 14)))


def kernel(batch_size, white_features, white_indices, black_features,
           black_indices, W1, b1, W2, b2, W3, b3):
    bs_res = (jnp.asarray(batch_size) - B).astype(jnp.float32)
    w1t = W1.reshape(NC, DH, -1).transpose(0, 2, 1)

    def chunked(a, b_):
        x = jnp.stack([a, b_]).reshape(NC, NCH_TOT, CH).astype(jnp.int32)
        return jnp.pad(x, ((0, 0), (0, GB), (0, 0)))

    feats = chunked(white_features, black_features)
    idxs = chunked(white_indices, black_indices)
    bnds = jnp.stack([_bounds(white_indices), _bounds(black_indices)])
    zblk = jnp.zeros((ROWS_PER_SUB, DH), jnp.float32)

    acc = _sc_accumulate(w1t, feats, idxs, bnds, zblk)

    out = _tail(acc, (b1 + bs_res).reshape(1, D), W2.T, b2.reshape(1, 32),
                W3, b3.reshape(1, 1))
    return out[:, 0]


# speedup vs baseline: 3.1756x; 1.2908x over previous
import functools

import jax
import jax.numpy as jnp
from jax import lax
from jax.experimental import pallas as pl
from jax.experimental.pallas import tpu as pltpu
from jax.experimental.pallas import tpu_sc as plsc

B = 4096
D = 256
DH = D // 2
NF = 131072
NC, NS = 2, 16
CH = 64
GB = 64
NCH_TOT = NF // CH
ROWS_PER_SUB = B // NS
TRASH_ROW = ROWS_PER_SUB
ACC_ROWS = ROWS_PER_SUB + 8


def _sc_body(w1t, feats, idxs, bnds, zblk, out, fbig, bbig, rows, bv, libuf,
             acc, gsem):
    c = lax.axis_index("c")
    s = lax.axis_index("s")

    def side(v):
        pltpu.sync_copy(zblk, acc)
        pltpu.sync_copy(bnds.at[v, s], bv)
        bvv = bv[...]
        lo = bvv[0]
        hi = bvv[1]
        a_lo = (lo // (8 * CH)) * (8 * CH)
        crow0 = a_lo // CH
        nch = (hi - a_lo + CH - 1) // CH

        def stage_bulk(g):
            gpar = lax.rem(g, 2)
            start = pl.multiple_of(crow0 + g * GB, 8)
            pltpu.sync_copy(feats.at[v, pl.ds(start, GB)], fbig.at[gpar])
            pltpu.sync_copy(idxs.at[v, pl.ds(start, GB)], bbig.at[gpar])

        def issue_gather(k):
            gpar = lax.rem(k // GB, 2)
            cpar = lax.rem(k, 2)
            pltpu.async_copy(w1t.at[c].at[fbig.at[gpar, lax.rem(k, GB)]],
                             rows.at[cpar], gsem)

        stage_bulk(0)

        @pl.when(nch > 0)
        def _():
            issue_gather(0)

        def chunk(k, carry):
            cpar = lax.rem(k, 2)
            gpar = lax.rem(k // GB, 2)
            km = lax.rem(k, GB)
            pltpu.make_async_copy(w1t.at[c].at[fbig.at[gpar, km]],
                                  rows.at[cpar], gsem).wait()

            @pl.when(k + 1 < nch)
            def _():
                @pl.when(lax.rem(k + 1, GB) == 0)
                def _():
                    stage_bulk((k + 1) // GB)
                issue_gather(k + 1)

            base = a_lo + k * CH
            iota = lax.iota(jnp.int32, 16)

            for t in range(CH // 16):
                sl16 = pl.ds(t * 16, 16)
                p = base + t * 16 + iota
                li = bbig[gpar, km, sl16] - s * ROWS_PER_SUB
                valid = jnp.logical_and(p >= lo, p < hi)
                liv = jnp.where(valid, li, TRASH_ROW)

                for rr in range(16):
                    li_r = liv[rr]
                    for j in range(DH // 16):
                        sl = pl.ds(j * 16, 16)
                        plsc.addupdate(acc.at[li_r, sl],
                                       rows[cpar, t * 16 + rr, sl])
            return carry

        lax.fori_loop(0, nch, chunk, 0)

        pltpu.sync_copy(
            acc.at[pl.ds(0, ROWS_PER_SUB)],
            out.at[v, pl.ds(s * ROWS_PER_SUB, ROWS_PER_SUB),
                   pl.ds(c * DH, DH)])

    side(0)
    side(1)


_sc_accumulate = functools.partial(
    pl.kernel,
    out_type=jax.ShapeDtypeStruct((NC, B, D), jnp.float32),
    mesh=plsc.VectorSubcoreMesh(
        core_axis_name="c", subcore_axis_name="s",
        num_cores=NC, num_subcores=NS),
    scratch_types=[
        pltpu.VMEM((2, GB, CH), jnp.int32),
        pltpu.VMEM((2, GB, CH), jnp.int32),
        pltpu.VMEM((2, CH, DH), jnp.float32),
        pltpu.VMEM((16,), jnp.int32),
        pltpu.VMEM((CH,), jnp.int32),
        pltpu.VMEM((ACC_ROWS, DH), jnp.float32),
        pltpu.SemaphoreType.DMA,
    ],
)(_sc_body)


def _tail_body(a1w, a1b, b1r, w2t, b2, w3, b3, out):
    h = jnp.concatenate(
        [jnp.clip(a1w[0] + b1r[...], 0.0, 1.0),
         jnp.clip(a1b[0] + b1r[...], 0.0, 1.0)], axis=1)
    a2 = jnp.dot(h, w2t[...], preferred_element_type=jnp.float32) + b2[...]
    a2 = jnp.clip(a2, 0.0, 1.0)
    out[...] = jnp.sum(a2 * w3[...], axis=1, keepdims=True) + b3[...]


def _tail(acc, b1r, w2t, b2, w3, b3):
    R = 512
    return pl.pallas_call(
        _tail_body,
        grid=(B // R,),
        in_specs=[
            pl.BlockSpec((1, R, D), lambda i: (0, i, 0)),
            pl.BlockSpec((1, R, D), lambda i: (1, i, 0)),
            pl.BlockSpec((1, D), lambda i: (0, 0)),
            pl.BlockSpec((2 * D, 32), lambda i: (0, 0)),
            pl.BlockSpec((1, 32), lambda i: (0, 0)),
            pl.BlockSpec((1, 32), lambda i: (0, 0)),
            pl.BlockSpec((1, 1), lambda i: (0, 0)),
        ],
        out_specs=pl.BlockSpec((R, 1), lambda i: (i, 0)),
        out_shape=jax.ShapeDtypeStruct((B, 1), jnp.float32),
    )(acc, acc, b1r, w2t, b2, w3, b3)


def _bounds(indices):
    edges = jnp.arange(0, B + 1, ROWS_PER_SUB, dtype=jnp.int32)
    b = jnp.searchsorted(indices, edges, side="left").astype(jnp.int32)
    lohi = jnp.stack([b[:-1], b[1:]], axis=-1)
    return jnp.pad(lohi, ((0, 0), (0, 14)))


def kernel(batch_size, white_features, white_indices, black_features,
           black_indices, W1, b1, W2, b2, W3, b3):
    bs_res = (jnp.asarray(batch_size) - B).astype(jnp.float32)
    w1t = W1.reshape(NC, DH, -1).transpose(0, 2, 1)

    def chunked(a, b_):
        x = jnp.stack([a, b_]).reshape(NC, NCH_TOT, CH).astype(jnp.int32)
        return jnp.pad(x, ((0, 0), (0, GB), (0, 0)))

    feats = chunked(white_features, black_features)
    idxs = chunked(white_indices, black_indices)
    bnds = jnp.stack([_bounds(white_indices), _bounds(black_indices)])
    zblk = jnp.zeros((ACC_ROWS, DH), jnp.float32)

    acc = _sc_accumulate(w1t, feats, idxs, bnds, zblk)

    out = _tail(acc, (b1 + bs_res).reshape(1, D), W2.T, b2.reshape(1, 32),
                W3, b3.reshape(1, 1))
    return out[:, 0]
